# baseline (speedup 1.0000x reference)
import functools

import jax
import jax.numpy as jnp
from jax import lax
from jax.experimental import pallas as pl
from jax.experimental.pallas import tpu as pltpu
from jax.experimental.pallas import tpu_sc as plsc

_VOCAB = 1000000
_MAXLEN = 200
_EMBED = 64
_BATCH = 4096

_NC = 2
_NS = 16
_NW = _NC * _NS
_SEQ_PER_W = _BATCH // _NW
_CHUNK_ROWS = _MAXLEN
_N_CHUNKS = _SEQ_PER_W
_SPLITS = ((0, 128), (128, 72))
_NSLOT = 2
_NIDX = 4


def _emb_body(x_hbm, tok_hbm, pos_hbm, out_hbm, idx_v, rows_v, obuf_v, pos_v,
              gsem, isem, osem):
    wid = lax.axis_index("s") * _NC + lax.axis_index("c")
    row_base = wid * _SEQ_PER_W * _MAXLEN

    def chunk_row0(g):
        return pl.multiple_of(row_base + g * _CHUNK_ROWS, 8)

    def idx_fire(g, u):
        return pltpu.async_copy(
            x_hbm.at[pl.ds(chunk_row0(g), _CHUNK_ROWS)], idx_v[u], isem[u])

    def idx_wait(u):
        pltpu.make_async_copy(
            x_hbm.at[pl.ds(0, _CHUNK_ROWS)], idx_v[u], isem[u]).wait()

    def gather_fire(u, t):
        for (o, n) in _SPLITS:
            pltpu.async_copy(
                tok_hbm.at[idx_v[u].at[pl.ds(o, n)]],
                rows_v[t].at[pl.ds(o, n), :],
                gsem[t])

    def gather_wait(u, t):
        for (o, n) in _SPLITS:
            pltpu.make_async_copy(
                tok_hbm.at[idx_v[u].at[pl.ds(o, n)]],
                rows_v[t].at[pl.ds(o, n), :],
                gsem[t]).wait()

    def wb_fire(g, t):
        pltpu.async_copy(
            obuf_v[t], out_hbm.at[pl.ds(chunk_row0(g), _CHUNK_ROWS), :],
            osem[t])

    def wb_wait(g, t):
        pltpu.make_async_copy(
            obuf_v[t], out_hbm.at[pl.ds(chunk_row0(g), _CHUNK_ROWS), :],
            osem[t]).wait()

    def add_pos(t):
        rows = rows_v[t]
        ob = obuf_v[t]

        def pbody(p, carry):
            for q in range(4):
                sl = pl.ds(q * 16, 16)
                ob[p, sl] = rows[p, sl] + pos_v[p, sl]
            return carry

        lax.fori_loop(0, _MAXLEN, pbody, 0)

    pltpu.sync_copy(pos_hbm, pos_v)

    idx_fire(0, 0).wait()
    idx_fire(1, 1).wait()
    gather_fire(0, 0)
    gather_fire(1, 1)
    idx_fire(2, 2)
    idx_fire(3, 3)

    n_outer = _N_CHUNKS // _NIDX

    def body(i, carry):
        for s in range(_NIDX):
            g = i * _NIDX + s
            rs = s % _NSLOT
            gather_wait(s, rs)
            if s < _NSLOT:
                @pl.when(i > 0)
                def _():
                    wb_wait(g - _NSLOT, rs)
            else:
                wb_wait(g - _NSLOT, rs)
            add_pos(rs)
            if s < _NSLOT:
                idx_wait((s + _NSLOT) % _NIDX)
                gather_fire((s + _NSLOT) % _NIDX, rs)
            else:
                @pl.when(i < n_outer - 1)
                def _():
                    idx_wait((s + _NSLOT) % _NIDX)
                    gather_fire((s + _NSLOT) % _NIDX, rs)
            @pl.when(i < n_outer - 1)
            def _():
                idx_fire(g + _NIDX, s)
            wb_fire(g, rs)
        return carry

    lax.fori_loop(0, n_outer, body, 0)

    for s in range(_NSLOT):
        wb_wait(_N_CHUNKS - _NSLOT + s, s)


_VCHUNK = 128
_NFULL = _VOCAB // _VCHUNK
_VREM = _VOCAB - _NFULL * _VCHUNK


def _tr_body(tokT_hbm, scr_hbm, blk_v, rows_v, sem_i, sem_o):
    wid = lax.axis_index("s") * _NC + lax.axis_index("c")
    iota = lax.iota(jnp.int32, 16)

    def in_fire(c, t):
        v0 = pl.multiple_of(c * _VCHUNK, _VCHUNK)
        return pltpu.async_copy(
            tokT_hbm.at[:, pl.ds(v0, _VCHUNK)], blk_v[t], sem_i[t])

    def in_wait(t):
        pltpu.make_async_copy(
            tokT_hbm.at[:, pl.ds(0, _VCHUNK)], blk_v[t], sem_i[t]).wait()

    def out_fire(c, t):
        v0 = pl.multiple_of(c * _VCHUNK, _VCHUNK)
        pltpu.async_copy(rows_v[t], scr_hbm.at[pl.ds(v0, _VCHUNK), :],
                         sem_o[t])

    def out_wait(c, t):
        pltpu.make_async_copy(
            rows_v[t], scr_hbm.at[pl.ds(0, _VCHUNK), :], sem_o[t]).wait()

    def transpose_blk(t, nv):
        blk = blk_v[t]
        rows = rows_v[t]

        def vbody(dv, carry):
            for q in range(4):
                idx = q * (16 * _VCHUNK) + iota * _VCHUNK + dv
                vals = plsc.load_gather(blk, [idx])
                rows[dv, pl.ds(q * 16, 16)] = vals
            return carry

        lax.fori_loop(0, nv, vbody, 0)

    in_fire(wid, 0)
    in_fire(wid + _NW, 1)

    def body(i, carry):
        for s in range(2):
            k = i * 2 + s
            c = wid + k * _NW
            in_wait(s)
            @pl.when(i > 0)
            def _():
                out_wait(c - 2 * _NW, s)
            transpose_blk(s, _VCHUNK)
            out_fire(c, s)
            @pl.when(i < 121)
            def _():
                in_fire(c + 2 * _NW, s)
        return carry

    lax.fori_loop(0, 122, body, 0)
    for s in range(2):
        out_wait(0, s)

    @pl.when(wid < 4)
    def _():
        c = _NFULL - 4 + wid
        in_fire(c, 0).wait()
        transpose_blk(0, _VCHUNK)
        out_fire(c, 0)
        out_wait(c, 0)

    @pl.when(wid == 4)
    def _():
        v0 = _NFULL * _VCHUNK
        pltpu.async_copy(
            tokT_hbm.at[:, pl.ds(v0, _VREM)],
            blk_v[0].at[:, pl.ds(0, _VREM)], sem_i[0]).wait()

        def vbody(dv, carry):
            for q in range(4):
                idx = q * (16 * _VCHUNK) + iota * _VCHUNK + dv
                vals = plsc.load_gather(blk_v[0], [idx])
                rows_v[0][dv, pl.ds(q * 16, 16)] = vals
            return carry

        lax.fori_loop(0, _VREM, vbody, 0)
        pltpu.async_copy(
            rows_v[0].at[pl.ds(0, _VREM), :],
            scr_hbm.at[pl.ds(v0, _VREM), :], sem_o[0]).wait()


@jax.jit
def _transpose_table(tokT):
    mesh = plsc.VectorSubcoreMesh(core_axis_name="c", subcore_axis_name="s")
    f = functools.partial(
        pl.kernel,
        out_type=jax.ShapeDtypeStruct((_VOCAB, 2 * _EMBED), jnp.float32),
        mesh=mesh,
        compiler_params=pltpu.CompilerParams(use_tc_tiling_on_sc=True),
        scratch_types=[
            [pltpu.VMEM((_EMBED * _VCHUNK,), jnp.float32)] * 2,
            [pltpu.VMEM((_VCHUNK, 2 * _EMBED), jnp.float32)] * 2,
            [pltpu.SemaphoreType.DMA] * 2,
            [pltpu.SemaphoreType.DMA] * 2,
        ],
    )(_tr_body)
    return f(tokT)


@jax.jit
def _emb(xf, tok128, pos_table):
    mesh = plsc.VectorSubcoreMesh(core_axis_name="c", subcore_axis_name="s")
    f = functools.partial(
        pl.kernel,
        out_type=jax.ShapeDtypeStruct((_BATCH * _MAXLEN, _EMBED), jnp.float32),
        mesh=mesh,
        compiler_params=pltpu.CompilerParams(use_tc_tiling_on_sc=True),
        scratch_types=[
            [pltpu.VMEM((_CHUNK_ROWS,), jnp.int32)] * _NIDX,
            [pltpu.VMEM((_CHUNK_ROWS, 2 * _EMBED), jnp.float32)] * _NSLOT,
            [pltpu.VMEM((_CHUNK_ROWS, _EMBED), jnp.float32)] * _NSLOT,
            pltpu.VMEM((_MAXLEN, _EMBED), jnp.float32),
            [pltpu.SemaphoreType.DMA] * _NSLOT,
            [pltpu.SemaphoreType.DMA] * _NIDX,
            [pltpu.SemaphoreType.DMA] * _NSLOT,
        ],
    )(_emb_body)
    return f(xf, tok128, pos_table)


def kernel(x, token_table, pos_table):
    xf = x.astype(jnp.int32).reshape(_BATCH * _MAXLEN)
    tok128 = jnp.pad(token_table, ((0, 0), (0, 2 * _EMBED - token_table.shape[1])))
    out = _emb(xf, tok128, pos_table)
    return out.reshape(_BATCH, _MAXLEN, _EMBED)


# speedup vs baseline: 1.0006x; 1.0006x over previous
import functools

import jax
import jax.numpy as jnp
from jax import lax
from jax.experimental import pallas as pl
from jax.experimental.pallas import tpu as pltpu
from jax.experimental.pallas import tpu_sc as plsc

_VOCAB = 1000000
_MAXLEN = 200
_EMBED = 64
_BATCH = 4096

_NC = 2
_NS = 16
_NW = _NC * _NS
_SEQ_PER_W = _BATCH // _NW
_CHUNK_ROWS = _MAXLEN
_N_CHUNKS = _SEQ_PER_W
_SPLITS = ((0, 128), (128, 72))
_NSLOT = 2
_NIDX = 4


def _emb_body(x_hbm, tok_hbm, pos_hbm, out_hbm, idx_v, rows_v, obuf_v, pos_v,
              gsem, isem, osem):
    wid = lax.axis_index("s") * _NC + lax.axis_index("c")
    row_base = wid * _SEQ_PER_W * _MAXLEN

    def chunk_row0(g):
        return pl.multiple_of(row_base + g * _CHUNK_ROWS, 8)

    def idx_fire(g, u):
        return pltpu.async_copy(
            x_hbm.at[pl.ds(chunk_row0(g), _CHUNK_ROWS)], idx_v[u], isem[u])

    def idx_wait(u):
        pltpu.make_async_copy(
            x_hbm.at[pl.ds(0, _CHUNK_ROWS)], idx_v[u], isem[u]).wait()

    def gather_fire(u, t):
        for (o, n) in _SPLITS:
            pltpu.async_copy(
                tok_hbm.at[idx_v[u].at[pl.ds(o, n)]],
                rows_v[t].at[pl.ds(o, n), :],
                gsem[t])

    def gather_wait(u, t):
        for (o, n) in _SPLITS:
            pltpu.make_async_copy(
                tok_hbm.at[idx_v[u].at[pl.ds(o, n)]],
                rows_v[t].at[pl.ds(o, n), :],
                gsem[t]).wait()

    def wb_fire(g, t):
        pltpu.async_copy(
            obuf_v[t], out_hbm.at[pl.ds(chunk_row0(g), _CHUNK_ROWS), :],
            osem[t])

    def wb_wait(g, t):
        pltpu.make_async_copy(
            obuf_v[t], out_hbm.at[pl.ds(chunk_row0(g), _CHUNK_ROWS), :],
            osem[t]).wait()

    def add_pos(t):
        rows = rows_v[t]
        ob = obuf_v[t]

        def pbody(p, carry):
            for q in range(4):
                sl = pl.ds(q * 16, 16)
                ob[p, sl] = rows[p, sl] + pos_v[p, sl]
            return carry

        lax.fori_loop(0, _MAXLEN, pbody, 0)

    pltpu.sync_copy(pos_hbm, pos_v)

    idx_fire(0, 0).wait()
    idx_fire(1, 1).wait()
    gather_fire(0, 0)
    gather_fire(1, 1)
    idx_fire(2, 2)
    idx_fire(3, 3)

    n_outer = _N_CHUNKS // _NIDX

    def body(i, carry):
        for s in range(_NIDX):
            g = i * _NIDX + s
            rs = s % _NSLOT
            gather_wait(s, rs)
            if s < _NSLOT:
                @pl.when(i > 0)
                def _():
                    wb_wait(g - _NSLOT, rs)
            else:
                wb_wait(g - _NSLOT, rs)
            add_pos(rs)
            if s < _NSLOT:
                idx_wait((s + _NSLOT) % _NIDX)
                gather_fire((s + _NSLOT) % _NIDX, rs)
            else:
                @pl.when(i < n_outer - 1)
                def _():
                    idx_wait((s + _NSLOT) % _NIDX)
                    gather_fire((s + _NSLOT) % _NIDX, rs)
            @pl.when(i < n_outer - 1)
            def _():
                idx_fire(g + _NIDX, s)
            wb_fire(g, rs)
        return carry

    lax.fori_loop(0, n_outer, body, 0)

    for s in range(_NSLOT):
        wb_wait(_N_CHUNKS - _NSLOT + s, s)


@jax.jit
def _emb(xf, tok128, pos_table):
    mesh = plsc.VectorSubcoreMesh(core_axis_name="c", subcore_axis_name="s")
    f = functools.partial(
        pl.kernel,
        out_type=jax.ShapeDtypeStruct((_BATCH * _MAXLEN, _EMBED), jnp.float32),
        mesh=mesh,
        compiler_params=pltpu.CompilerParams(use_tc_tiling_on_sc=True),
        scratch_types=[
            [pltpu.VMEM((_CHUNK_ROWS,), jnp.int32)] * _NIDX,
            [pltpu.VMEM((_CHUNK_ROWS, 2 * _EMBED), jnp.float32)] * _NSLOT,
            [pltpu.VMEM((_CHUNK_ROWS, _EMBED), jnp.float32)] * _NSLOT,
            pltpu.VMEM((_MAXLEN, _EMBED), jnp.float32),
            [pltpu.SemaphoreType.DMA] * _NSLOT,
            [pltpu.SemaphoreType.DMA] * _NIDX,
            [pltpu.SemaphoreType.DMA] * _NSLOT,
        ],
    )(_emb_body)
    return f(xf, tok128, pos_table)


def kernel(x, token_table, pos_table):
    xf = x.astype(jnp.int32).reshape(_BATCH * _MAXLEN)
    tok128 = jnp.pad(token_table, ((0, 0), (0, 2 * _EMBED - token_table.shape[1])))
    out = _emb(xf, tok128, pos_table)
    return out.reshape(_BATCH, _MAXLEN, _EMBED)
